# baseline (speedup 1.0000x reference)
import math

import jax
import jax.numpy as jnp
import numpy as np
from jax.experimental import pallas as pl

_SCALE = 1.0 / math.sqrt(12.0)
_ROWS = 32768
_COLS = 384
_TOK = 32
_BLK = 512


def _body(x_ref, w_ref, q_ref, t_ref):
    x = x_ref[...]
    s = jnp.where(x >= 0.0, jnp.float32(1.0), jnp.float32(-1.0))
    q_ref[...] = s * jnp.float32(_SCALE)
    d = jnp.dot(s, w_ref[...], preferred_element_type=jnp.float32)
    t_ref[...] = ((d + 4095.0) * 0.5).astype(jnp.int32)


_W = np.zeros((_COLS, _TOK), np.float32)
for _c in range(_COLS):
    _W[_c, _c // 12] = float(2 ** (_c % 12))
_WJ = jnp.asarray(_W)


def kernel(latents):
    xv = latents.reshape(_ROWS, _COLS)
    q, t = pl.pallas_call(
        _body,
        grid=(_ROWS // _BLK,),
        in_specs=[
            pl.BlockSpec((_BLK, _COLS), lambda i: (i, 0)),
            pl.BlockSpec((_COLS, _TOK), lambda i: (0, 0)),
        ],
        out_specs=[
            pl.BlockSpec((_BLK, _COLS), lambda i: (i, 0)),
            pl.BlockSpec((_BLK, _TOK), lambda i: (i, 0)),
        ],
        out_shape=[
            jax.ShapeDtypeStruct((_ROWS, _COLS), jnp.float32),
            jax.ShapeDtypeStruct((_ROWS, _TOK), jnp.int32),
        ],
    )(xv, _WJ)
    quant = q.reshape(1024, 1024, 12)
    tokens = t.reshape(1024, 1024).astype(jnp.int64)
    return (quant, tokens)


# speedup vs baseline: 3.7802x; 3.7802x over previous
import math

import jax
import jax.numpy as jnp
import numpy as np
from jax.experimental import pallas as pl

_SCALE = 1.0 / math.sqrt(12.0)
_R = 1024
_C = 12288
_SEG = 384
_NSEG = _C // _SEG
_TSEG = 32
_BLK = 128


def _body(x_ref, w_ref, q_ref, t_ref):
    x = x_ref[...]
    s = jnp.where(x >= 0.0, jnp.float32(1.0), jnp.float32(-1.0))
    q_ref[...] = s * jnp.float32(_SCALE)
    sb = s.astype(jnp.bfloat16)
    w = w_ref[...]
    for g in range(_NSEG):
        seg = jax.lax.slice_in_dim(sb, _SEG * g, _SEG * (g + 1), axis=1)
        d = jax.lax.dot_general(seg, w, (((1,), (0,)), ((), ())),
                                preferred_element_type=jnp.float32)
        t_ref[:, _TSEG * g:_TSEG * (g + 1)] = ((d + 4095.0) * 0.5).astype(jnp.int32)


_W = np.zeros((_SEG, _TSEG), np.float32)
for _c in range(_SEG):
    _W[_c, _c // 12] = float(2 ** (_c % 12))
def kernel(latents):
    xv = latents.reshape(_R, _C)
    q, t = pl.pallas_call(
        _body,
        grid=(_R // _BLK,),
        in_specs=[
            pl.BlockSpec((_BLK, _C), lambda i: (i, 0)),
            pl.BlockSpec((_SEG, _TSEG), lambda i: (0, 0)),
        ],
        out_specs=[
            pl.BlockSpec((_BLK, _C), lambda i: (i, 0)),
            pl.BlockSpec((_BLK, _R), lambda i: (i, 0)),
        ],
        out_shape=[
            jax.ShapeDtypeStruct((_R, _C), jnp.float32),
            jax.ShapeDtypeStruct((_R, _R), jnp.int32),
        ],
    )(xv, jnp.asarray(_W, dtype=jnp.bfloat16))
    quant = q.reshape(1024, 1024, 12)
    tokens = t.astype(jnp.int64)
    return (quant, tokens)


# speedup vs baseline: 34.1238x; 9.0270x over previous
import math

import jax
import jax.numpy as jnp
from jax.experimental import pallas as pl

_SCALE = 1.0 / math.sqrt(12.0)
_L = 12
_N = 1024
_BLK = 64


def _body(x_ref, q_ref, t_ref):
    acc = None
    for k in range(_L):
        m = x_ref[k] >= 0.0
        q_ref[k] = jnp.where(m, jnp.float32(_SCALE), jnp.float32(-_SCALE))
        term = jnp.where(m, jnp.int32(1 << k), jnp.int32(0))
        acc = term if acc is None else acc + term
    t_ref[...] = acc


def kernel(latents):
    xt = jnp.transpose(latents, (2, 0, 1))
    q3, t = pl.pallas_call(
        _body,
        grid=(_N // _BLK,),
        in_specs=[
            pl.BlockSpec((_L, _BLK, _N), lambda i: (0, i, 0)),
        ],
        out_specs=[
            pl.BlockSpec((_L, _BLK, _N), lambda i: (0, i, 0)),
            pl.BlockSpec((_BLK, _N), lambda i: (i, 0)),
        ],
        out_shape=[
            jax.ShapeDtypeStruct((_L, _N, _N), jnp.float32),
            jax.ShapeDtypeStruct((_N, _N), jnp.int32),
        ],
    )(xt)
    quant = jnp.transpose(q3, (1, 2, 0))
    tokens = t.astype(jnp.int64)
    return (quant, tokens)


# speedup vs baseline: 34.9790x; 1.0251x over previous
import math

import jax
import jax.numpy as jnp
from jax.experimental import pallas as pl

_SCALE = 1.0 / math.sqrt(12.0)
_L = 12
_N = 1024
_BLK = 128


def _body(x_ref, q_ref, t_ref):
    acc = None
    for k in range(_L):
        m = x_ref[k] >= 0.0
        q_ref[k] = jnp.where(m, jnp.float32(_SCALE), jnp.float32(-_SCALE))
        term = jnp.where(m, jnp.int32(1 << k), jnp.int32(0))
        acc = term if acc is None else acc + term
    t_ref[...] = acc


def kernel(latents):
    xt = jnp.transpose(latents, (2, 0, 1))
    q3, t = pl.pallas_call(
        _body,
        grid=(_N // _BLK,),
        in_specs=[
            pl.BlockSpec((_L, _BLK, _N), lambda i: (0, i, 0)),
        ],
        out_specs=[
            pl.BlockSpec((_L, _BLK, _N), lambda i: (0, i, 0)),
            pl.BlockSpec((_BLK, _N), lambda i: (i, 0)),
        ],
        out_shape=[
            jax.ShapeDtypeStruct((_L, _N, _N), jnp.float32),
            jax.ShapeDtypeStruct((_N, _N), jnp.int32),
        ],
    )(xt)
    quant = jnp.transpose(q3, (1, 2, 0))
    tokens = t.astype(jnp.int64)
    return (quant, tokens)


# speedup vs baseline: 37.6670x; 1.0768x over previous
import math

import jax
import jax.numpy as jnp
from jax.experimental import pallas as pl

_SCALE = 1.0 / math.sqrt(12.0)
_L = 12
_N = 1024
_BLK = 256


def _body(x_ref, q_ref, t_ref):
    acc = None
    for k in range(_L):
        m = x_ref[k] >= 0.0
        q_ref[k] = jnp.where(m, jnp.float32(_SCALE), jnp.float32(-_SCALE))
        term = jnp.where(m, jnp.int32(1 << k), jnp.int32(0))
        acc = term if acc is None else acc + term
    t_ref[...] = acc


def kernel(latents):
    xt = jnp.transpose(latents, (2, 0, 1))
    q3, t = pl.pallas_call(
        _body,
        grid=(_N // _BLK,),
        in_specs=[
            pl.BlockSpec((_L, _BLK, _N), lambda i: (0, i, 0)),
        ],
        out_specs=[
            pl.BlockSpec((_L, _BLK, _N), lambda i: (0, i, 0)),
            pl.BlockSpec((_BLK, _N), lambda i: (i, 0)),
        ],
        out_shape=[
            jax.ShapeDtypeStruct((_L, _N, _N), jnp.float32),
            jax.ShapeDtypeStruct((_N, _N), jnp.int32),
        ],
    )(xt)
    quant = jnp.transpose(q3, (1, 2, 0))
    tokens = t.astype(jnp.int64)
    return (quant, tokens)
